# Initial kernel scaffold; baseline (speedup 1.0000x reference)
import jax
import jax.numpy as jnp
from jax.experimental import pallas as pl


def kernel(occ_val_grid, pts, occ_val):
    raise NotImplementedError("write your pallas kernel here")


# speedup vs baseline: 1.4151x; 1.4151x over previous
import functools

import jax
import jax.numpy as jnp
from jax import lax
from jax.experimental import pallas as pl
from jax.experimental.pallas import tpu as pltpu
from jax.experimental.pallas import tpu_sc as plsc

DECAY = 0.95
RES = 256
N_PTS = 1048576
NVOX = RES * RES * RES

NW = 32
LANES = 16
NBIN = 512
VB = NVOX // NBIN
BIN_SHIFT = 15
LMASK = VB - 1
NP_W = N_PTS // NW
C1 = 2048
C2 = 2048
C3 = 1024
NPAD = N_PTS + 4096
SPAD = 544

_mesh = lambda: plsc.VectorSubcoreMesh(core_axis_name="c", subcore_axis_name="s")


def _wid():
    return lax.axis_index("s") * 2 + lax.axis_index("c")


def _iota16():
    return lax.broadcasted_iota(jnp.int32, (LANES,), 0)


def _k1_body(pts_hbm, key_hbm, counts_hbm, pbuf, kbuf, hist, cbuf):
    wid = _wid()
    iota = _iota16()

    def zb(i, _):
        hist[pl.ds(i * 16, 16)] = jnp.zeros((16,), jnp.int32)
        return 0

    lax.fori_loop(0, NBIN * 16 // 16, zb, 0)

    base_pt = wid * NP_W

    def chunk(qc, _):
        base = base_pt + qc * C1
        pltpu.sync_copy(pts_hbm.at[pl.ds(pl.multiple_of(base * 3, 8), C1 * 3)], pbuf)

        def vec(j, _):
            pi = (j * 16 + iota) * 3
            xs = plsc.load_gather(pbuf, [pi])
            ys = plsc.load_gather(pbuf, [pi + 1])
            zs = plsc.load_gather(pbuf, [pi + 2])

            def gcoord(t):
                tt = (t * 0.5 + 0.5) * 256.0
                return jnp.clip(tt.astype(jnp.int32), 0, RES - 1)

            key = gcoord(xs) * 65536 + gcoord(ys) * 256 + gcoord(zs)
            kbuf[pl.ds(j * 16, 16)] = key
            b = key >> BIN_SHIFT
            plsc.addupdate_scatter(hist, [iota * NBIN + b],
                                   jnp.ones((16,), jnp.int32))
            return 0

        lax.fori_loop(0, C1 // 16, vec, 0)
        pltpu.sync_copy(kbuf, key_hbm.at[pl.ds(pl.multiple_of(base, 8), C1)])
        return 0

    lax.fori_loop(0, NP_W // C1, chunk, 0)

    def red(bv, _):
        def add_t(t, acc):
            return acc + hist[pl.ds(t * NBIN + bv * 16, 16)]

        acc = lax.fori_loop(0, 16, add_t, jnp.zeros((16,), jnp.int32))
        cbuf[pl.ds(bv * 16, 16)] = acc
        return 0

    lax.fori_loop(0, NBIN // 16, red, 0)
    pltpu.sync_copy(cbuf, counts_hbm.at[pl.ds(pl.multiple_of(wid * NBIN, 8), NBIN)])


def _k2_body(key_hbm, val_hbm, counts_hbm, bkey_hbm, bval_hbm, starts_hbm,
             cntbuf, off, sbuf, kbuf, vbuf, dbuf, sem_k, sem_v):
    wid = _wid()
    iota = _iota16()

    pltpu.sync_copy(counts_hbm, cntbuf)

    def per_bv(bv, carry):
        def acc_w(w2, tp):
            tot, part = tp
            row = cntbuf[pl.ds(w2 * NBIN + bv * 16, 16)]
            part = part + jnp.where(w2 < wid, row, jnp.zeros((16,), jnp.int32))
            return tot + row, part

        tot, part = lax.fori_loop(
            0, NW, acc_w,
            (jnp.zeros((16,), jnp.int32), jnp.zeros((16,), jnp.int32)))
        inc = plsc.cumsum(tot)
        exc = inc - tot + carry
        off[pl.ds(bv * 16, 16)] = exc + part
        sbuf[pl.ds(bv * 16, 16)] = exc
        return carry + jnp.sum(tot)

    total = lax.fori_loop(0, NBIN // 16, per_bv, jnp.int32(0))
    sbuf[pl.ds(NBIN, 16)] = jnp.where(iota == 0, total, 0)
    sbuf[pl.ds(NBIN + 16, 16)] = jnp.zeros((16,), jnp.int32)

    @pl.when(wid == 0)
    def _():
        pltpu.sync_copy(sbuf, starts_hbm)

    def chunk(qc, _):
        rowbase = wid * (NP_W // 128) + qc * (C2 // 128)
        pltpu.sync_copy(key_hbm.at[pl.ds(rowbase, C2 // 128)], kbuf)
        pltpu.sync_copy(val_hbm.at[pl.ds(rowbase, C2 // 128)], vbuf)

        def row(r, _):
            for cc in range(128 // 16):
                kv = kbuf[r, pl.ds(cc * 16, 16)]
                b = kv >> BIN_SHIFT
                cntv, lastm = plsc.scan_count(b)
                offv = plsc.load_gather(off, [b])
                dest = offv + cntv - 1
                plsc.store_scatter(off, [b], dest + 1, mask=lastm)
                dbuf[r, pl.ds(cc * 16, 16)] = jnp.clip(dest, 0, NPAD - 1)
            return 0

        lax.fori_loop(0, C2 // 128, row, 0)
        hs = []
        for r in range(C2 // 128):
            hs.append(pltpu.async_copy(kbuf.at[r], bkey_hbm.at[dbuf.at[r]], sem_k))
            hs.append(pltpu.async_copy(vbuf.at[r], bval_hbm.at[dbuf.at[r]], sem_v))
        for h in hs:
            h.wait()
        return 0

    lax.fori_loop(0, NP_W // C2, chunk, 0)


def _k3_body(grid_hbm, bkey_hbm, bval_hbm, starts_hbm, out_hbm,
             sorig, obuf, kch, vch, stv):
    wid = _wid()
    iota = _iota16()

    pltpu.sync_copy(starts_hbm, stv)

    def bin_i(i, _):
        b = i * NW + wid
        vb_base = pl.multiple_of(b * VB, 8)
        gidx = b + jnp.minimum(iota, 1)
        sv = plsc.load_gather(stv, [gidx])
        start = jnp.max(jnp.where(iota == 0, sv, 0))
        end = jnp.max(jnp.where(iota == 1, sv, 0))
        cnt = end - start
        pltpu.sync_copy(grid_hbm.at[pl.ds(vb_base, VB)], sorig)
        pltpu.sync_copy(grid_hbm.at[pl.ds(vb_base, VB)], obuf)
        astart = pl.multiple_of((start // 8) * 8, 8)
        nch = jnp.clip((end - astart + (C3 - 1)) // C3, 0, NPAD // C3)

        def chA(q, _):
            cb = pl.multiple_of(astart + q * C3, 8)
            pltpu.sync_copy(bkey_hbm.at[pl.ds(cb, C3)], kch)

            def vA(j, _):
                kv = kch[pl.ds(j * 16, 16)]
                gpos = cb + j * 16 + iota
                valid = (gpos >= start) & (gpos < end)
                lidx = jnp.where(valid, kv & LMASK, 0)
                g = plsc.load_gather(sorig, [lidx])
                plsc.store_scatter(obuf, [lidx], g * DECAY, mask=valid)
                return 0

            lax.fori_loop(0, C3 // 16, vA, 0)
            return 0

        lax.fori_loop(0, nch, chA, 0)

        def chB(q, _):
            cb = pl.multiple_of(astart + q * C3, 8)
            pltpu.sync_copy(bkey_hbm.at[pl.ds(cb, C3)], kch)
            pltpu.sync_copy(bval_hbm.at[pl.ds(cb, C3)], vch)

            def vB(j, _):
                kv = kch[pl.ds(j * 16, 16)]
                vv = vch[pl.ds(j * 16, 16)]
                gpos = cb + j * 16 + iota
                valid = (gpos >= start) & (gpos < end)
                lidx = jnp.where(valid, kv & LMASK, 0)
                cntv, _ = plsc.scan_count(lidx)
                rounds = jnp.max(jnp.where(valid, cntv, 0)) + 1

                def rnd(r, _):
                    sel = valid & (cntv == r)
                    g = plsc.load_gather(obuf, [lidx])
                    plsc.store_scatter(obuf, [lidx], jnp.maximum(g, vv),
                                       mask=sel)
                    return 0

                lax.fori_loop(0, rounds, rnd, 0)
                return 0

            lax.fori_loop(0, C3 // 16, vB, 0)
            return 0

        lax.fori_loop(0, nch, chB, 0)
        pltpu.sync_copy(obuf, out_hbm.at[pl.ds(vb_base, VB)])
        return 0

    lax.fori_loop(0, NBIN // NW, bin_i, 0)


def _make_kernels():
    cp = pltpu.CompilerParams(needs_layout_passes=False)
    k1 = functools.partial(
        pl.kernel, _k1_body, mesh=_mesh(),
        out_type=(jax.ShapeDtypeStruct((N_PTS,), jnp.int32),
                  jax.ShapeDtypeStruct((NW * NBIN,), jnp.int32)),
        scratch_types=[pltpu.VMEM((C1 * 3,), jnp.float32),
                       pltpu.VMEM((C1,), jnp.int32),
                       pltpu.VMEM((16 * NBIN,), jnp.int32),
                       pltpu.VMEM((NBIN,), jnp.int32)],
        compiler_params=cp, name="occ_k1_keys_hist")()
    k2 = functools.partial(
        pl.kernel, _k2_body, mesh=_mesh(),
        out_type=(jax.ShapeDtypeStruct((NPAD,), jnp.int32),
                  jax.ShapeDtypeStruct((NPAD,), jnp.float32),
                  jax.ShapeDtypeStruct((SPAD,), jnp.int32)),
        scratch_types=[pltpu.VMEM((NW * NBIN,), jnp.int32),
                       pltpu.VMEM((NBIN,), jnp.int32),
                       pltpu.VMEM((SPAD,), jnp.int32),
                       pltpu.VMEM((C2 // 128, 128), jnp.int32),
                       pltpu.VMEM((C2 // 128, 128), jnp.float32),
                       pltpu.VMEM((C2 // 128, 128), jnp.int32),
                       pltpu.SemaphoreType.DMA,
                       pltpu.SemaphoreType.DMA],
        compiler_params=cp, name="occ_k2_route")()
    k3 = functools.partial(
        pl.kernel, _k3_body, mesh=_mesh(),
        out_type=jax.ShapeDtypeStruct((NVOX,), jnp.float32),
        scratch_types=[pltpu.VMEM((VB,), jnp.float32),
                       pltpu.VMEM((VB,), jnp.float32),
                       pltpu.VMEM((C3,), jnp.int32),
                       pltpu.VMEM((C3,), jnp.float32),
                       pltpu.VMEM((SPAD,), jnp.int32)],
        compiler_params=cp, name="occ_k3_slab_update")()
    return k1, k2, k3


_K1, _K2, _K3 = _make_kernels()


def kernel(occ_val_grid, pts, occ_val):
    grid_flat = occ_val_grid.reshape(-1)
    pts_flat = pts.reshape(-1)
    keys, counts = _K1(pts_flat)
    key2d = keys.reshape(N_PTS // 128, 128)
    val2d = occ_val.reshape(N_PTS // 128, 128)
    bkey, bval, starts = _K2(key2d, val2d, counts)
    out = _K3(grid_flat, bkey, bval, starts)
    return out.reshape(RES, RES, RES)
